# baseline (speedup 1.0000x reference)
import jax
import jax.numpy as jnp
from jax import lax
from jax.experimental import pallas as pl
from jax.experimental.pallas import tpu as pltpu

N = 5000
BLK = 512
NBLK = 10
NP = BLK * NBLK
NR = 16
NC = 512
K = 100
SIGMA = 0.5
SCORE_THRESH = 0.05


def _r2(f, x):
    return f(f(x, axis=0, keepdims=True), axis=1, keepdims=True)


def _nms_kernel(xc1, yc1, xc2, yc2, sc,
                xr1, yr1, xr2, yr2, sr,
                det_ref, dmax_ref):
    jb = pl.program_id(0)

    @pl.when(jb == 0)
    def _init():
        dmax_ref[...] = jnp.zeros((NR, NC), jnp.float32)

    x1r = xr1[pl.ds(jb, 1), :]
    y1r = yr1[pl.ds(jb, 1), :]
    x2r = xr2[pl.ds(jb, 1), :]
    y2r = yr2[pl.ds(jb, 1), :]
    srj = sr[pl.ds(jb, 1), :]
    arj = (x2r - x1r) * (y2r - y1r)
    jj = lax.broadcasted_iota(jnp.int32, (1, NC), 1) + jb * BLK

    acc = jnp.zeros((1, NC), jnp.float32)
    for ib in range(NBLK):
        rs = pl.ds(ib * BLK, BLK)
        x1c = xc1[rs, :]
        y1c = yc1[rs, :]
        x2c = xc2[rs, :]
        y2c = yc2[rs, :]
        scb = sc[rs, :]
        ac = (x2c - x1c) * (y2c - y1c)
        xx1 = jnp.maximum(x1c, x1r)
        yy1 = jnp.maximum(y1c, y1r)
        xx2 = jnp.minimum(x2c, x2r)
        yy2 = jnp.minimum(y2c, y2r)
        iw = jnp.maximum(xx2 - xx1, 0.0)
        ih = jnp.maximum(yy2 - yy1, 0.0)
        inter = iw * ih
        union = ac + arj - inter
        iou = inter / (union + 1e-8)
        ii = lax.broadcasted_iota(jnp.int32, (BLK, 1), 0) + ib * BLK
        m = (scb > srj) | ((scb == srj) & (ii < jj))
        acc = jnp.maximum(
            acc, jnp.max(jnp.where(m, iou, 0.0), axis=0, keepdims=True))
    dmax_ref[pl.ds(jb, 1), :] = acc

    @pl.when(jb == NBLK - 1)
    def _phase2():
        m_all = dmax_ref[...]
        s_all = sr[...]
        x1a = xr1[...]
        y1a = yr1[...]
        x2a = xr2[...]
        y2a = yr2[...]
        valid = s_all > -0.5
        draw = s_all * jnp.exp(-(m_all * m_all) / SIGMA)
        dthr = jnp.where(draw > SCORE_THRESH, draw, 0.0)
        d0 = jnp.where(valid, dthr, -1.0)
        idxf = (lax.broadcasted_iota(jnp.int32, (NR, NC), 0) * NC
                + lax.broadcasted_iota(jnp.int32, (NR, NC), 1)
                ).astype(jnp.float32)

        def body(k, carry):
            d, out = carry
            mv = _r2(jnp.max, d)
            t1 = d == mv
            sm = _r2(jnp.max, jnp.where(t1, s_all, -2.0))
            t2 = t1 & (s_all == sm)
            im = _r2(jnp.min, jnp.where(t2, idxf, 3.0e7))
            oh = t2 & (idxf == im)
            ohf = oh.astype(jnp.float32)
            vx1 = _r2(jnp.sum, ohf * x1a)
            vy1 = _r2(jnp.sum, ohf * y1a)
            vx2 = _r2(jnp.sum, ohf * x2a)
            vy2 = _r2(jnp.sum, ohf * y2a)
            rowi = lax.broadcasted_iota(jnp.int32, (8, 128), 0)
            lane = lax.broadcasted_iota(jnp.int32, (8, 128), 1)
            colv = jnp.where(rowi == 0, vx1,
                   jnp.where(rowi == 1, vy1,
                   jnp.where(rowi == 2, vx2,
                   jnp.where(rowi == 3, vy2,
                   jnp.where(rowi == 4, mv, 0.0)))))
            out = out + jnp.where(lane == k, colv, 0.0)
            d = jnp.where(oh, -2.0, d)
            return d, out

        _, out = lax.fori_loop(
            0, K, body, (d0, jnp.zeros((8, 128), jnp.float32)))
        det_ref[...] = out


def kernel(boxes, scores):
    boxes = boxes.astype(jnp.float32)
    scores = scores.astype(jnp.float32)
    total = NR * NC
    pad = total - N
    zpad = jnp.zeros((pad,), jnp.float32)
    s_pad = jnp.concatenate([scores, jnp.full((pad,), -1.0, jnp.float32)])
    x1 = jnp.concatenate([boxes[:, 0], zpad])
    y1 = jnp.concatenate([boxes[:, 1], zpad])
    x2 = jnp.concatenate([boxes[:, 2], zpad])
    y2 = jnp.concatenate([boxes[:, 3], zpad])

    def row(v):
        return v.reshape(NR, NC)

    def colm(v):
        return v[:NP, None]

    cspec = pl.BlockSpec((NP, 1), lambda j: (0, 0))
    rspec = pl.BlockSpec((NR, NC), lambda j: (0, 0))
    out = pl.pallas_call(
        _nms_kernel,
        grid=(NBLK,),
        in_specs=[cspec] * 5 + [rspec] * 5,
        out_specs=pl.BlockSpec((8, 128), lambda j: (0, 0)),
        out_shape=jax.ShapeDtypeStruct((8, 128), jnp.float32),
        scratch_shapes=[pltpu.VMEM((NR, NC), jnp.float32)],
    )(colm(x1), colm(y1), colm(x2), colm(y2), colm(s_pad),
      row(x1), row(y1), row(x2), row(y2), row(s_pad))
    return out[:5, :K].T
